# baseline (speedup 1.0000x reference)
import functools

import jax
import jax.numpy as jnp
from jax import lax
from jax.experimental import pallas as pl
from jax.experimental.pallas import tpu as pltpu
from jax.experimental.pallas import tpu_sc as plsc

VOCAB = 1000000
HIDDEN = 64
BATCH = 4096
SEQ = 50

NC = 2
NS = 16
NW = NC * NS

B_PER_W = BATCH // NW
G = 8
NG = B_PER_W // G


def _make_kernel():
    mesh = plsc.VectorSubcoreMesh(core_axis_name="c", subcore_axis_name="s")

    @functools.partial(
        pl.kernel,
        out_type=jax.ShapeDtypeStruct((BATCH, SEQ, HIDDEN), jnp.float32),
        mesh=mesh,
        scratch_types=[
            pltpu.VMEM((B_PER_W, SEQ), jnp.int32),
            pltpu.VMEM((G, SEQ, HIDDEN), jnp.float32),
            pltpu.VMEM((G, SEQ, HIDDEN), jnp.float32),
            pltpu.SemaphoreType.DMA,
            pltpu.SemaphoreType.DMA,
            pltpu.SemaphoreType.DMA,
            pltpu.SemaphoreType.DMA,
        ],
        compiler_params=pltpu.CompilerParams(use_tc_tiling_on_sc=False),
    )
    def embed(ids_hbm, table_hbm, out_hbm, idx_v, rows0, rows1, g0, g1, o0, o1):
        wid = lax.axis_index("s") * NC + lax.axis_index("c")
        base = wid * B_PER_W
        pltpu.sync_copy(ids_hbm.at[pl.ds(base, B_PER_W)], idx_v)

        bufs = (rows0, rows1)
        gsems = (g0, g1)
        osems = (o0, o1)

        def issue_gathers(s, b):
            return [
                pltpu.async_copy(
                    table_hbm.at[idx_v.at[s * G + k]],
                    bufs[b].at[k],
                    gsems[b])
                for k in range(G)
            ]

        pend_g = [issue_gathers(0, 0), issue_gathers(1, 1)]
        pend_o = [None, None]
        for s in range(NG):
            b = s % 2
            for h in pend_g[b]:
                h.wait()
            pend_o[b] = pltpu.async_copy(
                bufs[b], out_hbm.at[pl.ds(base + s * G, G)], osems[b])
            if s + 2 < NG:
                pend_o[b].wait()
                pend_g[b] = issue_gathers(s + 2, b)
        pend_o[NG % 2].wait()
        pend_o[(NG - 1) % 2].wait()

    return embed


_EMBED = _make_kernel()


def kernel(input_ids, word_embeddings):
    return _EMBED(input_ids.astype(jnp.int32), word_embeddings)
